# baseline (speedup 1.0000x reference)
import functools

import jax
import jax.numpy as jnp
from jax import lax
from jax.experimental import pallas as pl
from jax.experimental.pallas import tpu as pltpu
from jax.experimental.pallas import tpu_sc as plsc

B, S, D = 4, 8192, 1024
NC, NS = 2, 16
NW = NC * NS
POS_PER_W = S // NW
C = 32
NJ = POS_PER_W // C
CHUNK = C * D


@functools.partial(
    pl.kernel,
    mesh=plsc.VectorSubcoreMesh(core_axis_name="c", subcore_axis_name="s"),
    out_type=jax.ShapeDtypeStruct((B * S * D,), jnp.float32),
    scratch_types=[
        pltpu.VMEM((CHUNK,), jnp.float32),
        pltpu.VMEM((CHUNK,), jnp.float32),
    ],
)
def _pe_add_sc(x_hbm, pe_hbm, out_hbm, xbuf, pebuf):
    wid = lax.axis_index("s") * NC + lax.axis_index("c")

    def j_body(j, carry):
        pstart = (wid * POS_PER_W + j * C) * D
        pltpu.sync_copy(pe_hbm.at[pl.ds(pstart, CHUNK)], pebuf)

        def b_body(b, carry):
            xstart = b * (S * D) + pstart
            pltpu.sync_copy(x_hbm.at[pl.ds(xstart, CHUNK)], xbuf)

            def add_body(i, carry):
                sl = pl.ds(i * 16, 16)
                xbuf[sl] = xbuf[sl] + pebuf[sl]
                return carry

            lax.fori_loop(0, CHUNK // 16, add_body, 0)
            pltpu.sync_copy(xbuf, out_hbm.at[pl.ds(xstart, CHUNK)])
            return carry

        lax.fori_loop(0, B, b_body, 0)
        return carry

    lax.fori_loop(0, NJ, j_body, 0)


def kernel(x, pos_embed):
    pe = pos_embed[:S].reshape(-1)
    out = _pe_add_sc(x.reshape(-1), pe)
    return out.reshape(B, S, D)


# speedup vs baseline: 1.6752x; 1.6752x over previous
import functools

import jax
import jax.numpy as jnp
from jax import lax
from jax.experimental import pallas as pl
from jax.experimental.pallas import tpu as pltpu
from jax.experimental.pallas import tpu_sc as plsc

B, S, D = 4, 8192, 1024
NC, NS = 2, 16
NW = NC * NS
POS_PER_W = S // NW
C = 16
NJ = POS_PER_W // C
CH = C * D
U = 16


@functools.partial(
    pl.kernel,
    mesh=plsc.VectorSubcoreMesh(core_axis_name="c", subcore_axis_name="s"),
    out_type=jax.ShapeDtypeStruct((B * S * D,), jnp.float32),
    scratch_types=[
        pltpu.VMEM((CH,), jnp.float32),
        pltpu.VMEM((CH,), jnp.float32),
        pltpu.VMEM((CH,), jnp.float32),
        pltpu.VMEM((CH,), jnp.float32),
        pltpu.VMEM((CH,), jnp.float32),
        pltpu.SemaphoreType.DMA,
        pltpu.SemaphoreType.DMA,
        pltpu.SemaphoreType.DMA,
        pltpu.SemaphoreType.DMA,
        pltpu.SemaphoreType.DMA,
    ],
)
def _pe_add_sc(x_hbm, pe_hbm, out_hbm, xb0, xb1, ob0, ob1, peb,
               si0, si1, so0, so1, spe):
    wid = lax.axis_index("s") * NC + lax.axis_index("c")
    base = wid * POS_PER_W * D

    def add_chunk(xb, ob):
        def body(i, carry):
            i0 = i * (16 * U)
            for u in range(U):
                sl = pl.ds(i0 + u * 16, 16)
                ob[sl] = xb[sl] + peb[sl]
            return carry
        lax.fori_loop(0, CH // (16 * U), body, 0)

    def j_body(j, carry):
        pstart = base + j * CH
        pe_cp = pltpu.async_copy(pe_hbm.at[pl.ds(pstart, CH)], peb, spe)
        in0 = pltpu.async_copy(x_hbm.at[pl.ds(pstart, CH)], xb0, si0)
        in1 = pltpu.async_copy(x_hbm.at[pl.ds(S * D + pstart, CH)], xb1, si1)
        pe_cp.wait()
        in0.wait()
        add_chunk(xb0, ob0)
        o0 = pltpu.async_copy(ob0, out_hbm.at[pl.ds(pstart, CH)], so0)
        in2 = pltpu.async_copy(x_hbm.at[pl.ds(2 * S * D + pstart, CH)], xb0, si0)
        in1.wait()
        add_chunk(xb1, ob1)
        o1 = pltpu.async_copy(ob1, out_hbm.at[pl.ds(S * D + pstart, CH)], so1)
        in3 = pltpu.async_copy(x_hbm.at[pl.ds(3 * S * D + pstart, CH)], xb1, si1)
        in2.wait()
        o0.wait()
        add_chunk(xb0, ob0)
        o2 = pltpu.async_copy(ob0, out_hbm.at[pl.ds(2 * S * D + pstart, CH)], so0)
        in3.wait()
        o1.wait()
        add_chunk(xb1, ob1)
        o3 = pltpu.async_copy(ob1, out_hbm.at[pl.ds(3 * S * D + pstart, CH)], so1)
        o2.wait()
        o3.wait()
        return carry

    lax.fori_loop(0, NJ, j_body, 0)


def kernel(x, pos_embed):
    pe = pos_embed[:S].reshape(-1)
    out = _pe_add_sc(x.reshape(-1), pe)
    return out.reshape(B, S, D)


# speedup vs baseline: 3.7086x; 2.2138x over previous
import functools

import jax
import jax.numpy as jnp
from jax import lax
from jax.experimental import pallas as pl
from jax.experimental.pallas import tpu as pltpu
from jax.experimental.pallas import tpu_sc as plsc

B, S, D = 4, 8192, 1024
NC, NS = 2, 16
NW = NC * NS
POS_PER_W = S // NW
C = 16
NJ = POS_PER_W // C


@functools.partial(
    pl.kernel,
    mesh=plsc.VectorSubcoreMesh(core_axis_name="c", subcore_axis_name="s"),
    out_type=jax.ShapeDtypeStruct((B, S, D), jnp.float32),
    scratch_types=[
        pltpu.VMEM((C, D), jnp.float32),
        pltpu.VMEM((C, D), jnp.float32),
        pltpu.VMEM((C, D), jnp.float32),
        pltpu.VMEM((C, D), jnp.float32),
        pltpu.VMEM((C, D), jnp.float32),
        pltpu.SemaphoreType.DMA,
        pltpu.SemaphoreType.DMA,
        pltpu.SemaphoreType.DMA,
        pltpu.SemaphoreType.DMA,
        pltpu.SemaphoreType.DMA,
    ],
)
def _pe_add_sc(x_hbm, pe_hbm, out_hbm, xb0, xb1, ob0, ob1, peb,
               si0, si1, so0, so1, spe):
    wid = lax.axis_index("s") * NC + lax.axis_index("c")
    row0 = wid * POS_PER_W

    def add_chunk(xb, ob):
        def body(r, carry):
            for u in range(D // 16):
                sl = pl.ds(u * 16, 16)
                ob[r, sl] = xb[r, sl] + peb[r, sl]
            return carry
        lax.fori_loop(0, C, body, 0)

    def j_body(j, carry):
        p0 = row0 + j * C
        pe_cp = pltpu.async_copy(pe_hbm.at[pl.ds(p0, C)], peb, spe)
        in0 = pltpu.async_copy(x_hbm.at[0, pl.ds(p0, C)], xb0, si0)
        in1 = pltpu.async_copy(x_hbm.at[1, pl.ds(p0, C)], xb1, si1)
        pe_cp.wait()
        in0.wait()
        add_chunk(xb0, ob0)
        o0 = pltpu.async_copy(ob0, out_hbm.at[0, pl.ds(p0, C)], so0)
        in2 = pltpu.async_copy(x_hbm.at[2, pl.ds(p0, C)], xb0, si0)
        in1.wait()
        add_chunk(xb1, ob1)
        o1 = pltpu.async_copy(ob1, out_hbm.at[1, pl.ds(p0, C)], so1)
        in3 = pltpu.async_copy(x_hbm.at[3, pl.ds(p0, C)], xb1, si1)
        in2.wait()
        o0.wait()
        add_chunk(xb0, ob0)
        o2 = pltpu.async_copy(ob0, out_hbm.at[2, pl.ds(p0, C)], so0)
        in3.wait()
        o1.wait()
        add_chunk(xb1, ob1)
        o3 = pltpu.async_copy(ob1, out_hbm.at[3, pl.ds(p0, C)], so1)
        o2.wait()
        o3.wait()
        return carry

    lax.fori_loop(0, NJ, j_body, 0)


def kernel(x, pos_embed):
    return _pe_add_sc(x, pos_embed[:S])
